# baseline (speedup 1.0000x reference)
import jax
import jax.numpy as jnp
from jax.experimental import pallas as pl


def _layer_kernel(adj_ref, full_ref, blk_ref, kg_ref, bg_ref, w_ref, out_ref):
    hi = jax.lax.Precision.HIGHEST
    a = jnp.dot(adj_ref[...], full_ref[...], precision=hi,
                preferred_element_type=jnp.float32)
    t = jnp.maximum(
        jnp.dot(a, w_ref[...], precision=hi,
                preferred_element_type=jnp.float32), 0.0)
    g = jax.nn.sigmoid(
        jnp.dot(blk_ref[...], kg_ref[...], precision=hi,
                preferred_element_type=jnp.float32)
        + bg_ref[...])
    out_ref[...] = g * t + (1.0 - g) * blk_ref[...]


def kernel(x, adj, kernel_gate, bias_gate, Weight_1, Weight_2):
    n, d = x.shape
    bg = bias_gate.reshape(1, d)
    bm = next(b for b in (400, 200, 80, 40, 16, 8, n) if n % b == 0)
    grid = (n // bm,)

    nd = jax.ShapeDtypeStruct((n, d), jnp.float32)
    row_spec = pl.BlockSpec((bm, d), lambda i: (i, 0))
    full_spec = pl.BlockSpec((n, d), lambda i: (0, 0))
    sq_spec = pl.BlockSpec((d, d), lambda i: (0, 0))
    bias_spec = pl.BlockSpec((1, d), lambda i: (0, 0))
    adj_spec = pl.BlockSpec((bm, n), lambda i: (i, 0))

    layer = pl.pallas_call(
        _layer_kernel,
        grid=grid,
        in_specs=[adj_spec, full_spec, row_spec, sq_spec, bias_spec, sq_spec],
        out_specs=row_spec,
        out_shape=nd,
    )

    hg1 = layer(adj, x, x, kernel_gate, bg, Weight_1)
    return layer(adj, hg1, hg1, kernel_gate, bg, Weight_2)


# speedup vs baseline: 2.9454x; 2.9454x over previous
import jax
import jax.numpy as jnp
from jax.experimental import pallas as pl
from jax.experimental.pallas import tpu as pltpu


def _hgcn_kernel(adj_ref, x_ref, kg_ref, bg_ref, w1_ref, w2_ref,
                 out_ref, hg1_ref, *, bm, nb):
    i = pl.program_id(0)
    j = jnp.where(i < nb, i, i - nb)
    rows = pl.ds(j * bm, bm)

    @pl.when(i < nb)
    def _layer1():
        a = jnp.dot(adj_ref[...], x_ref[...],
                    preferred_element_type=jnp.float32)
        t = jnp.maximum(
            jnp.dot(a, w1_ref[...], preferred_element_type=jnp.float32), 0.0)
        x_blk = x_ref[rows, :]
        g = jax.nn.sigmoid(
            jnp.dot(x_blk, kg_ref[...], preferred_element_type=jnp.float32)
            + bg_ref[...])
        hg1_ref[rows, :] = g * t + (1.0 - g) * x_blk

    @pl.when(i >= nb)
    def _layer2():
        a = jnp.dot(adj_ref[...], hg1_ref[...],
                    preferred_element_type=jnp.float32)
        t = jnp.maximum(
            jnp.dot(a, w2_ref[...], preferred_element_type=jnp.float32), 0.0)
        h_blk = hg1_ref[rows, :]
        g = jax.nn.sigmoid(
            jnp.dot(h_blk, kg_ref[...], preferred_element_type=jnp.float32)
            + bg_ref[...])
        out_ref[...] = g * t + (1.0 - g) * h_blk


def kernel(x, adj, kernel_gate, bias_gate, Weight_1, Weight_2):
    n, d = x.shape
    bg = bias_gate.reshape(1, d)
    bm = next(b for b in (400, 200, 80, 40, 16, 8, n) if n % b == 0)
    nb = n // bm

    import functools
    body = functools.partial(_hgcn_kernel, bm=bm, nb=nb)

    full_spec = pl.BlockSpec((n, d), lambda i: (0, 0))
    sq_spec = pl.BlockSpec((d, d), lambda i: (0, 0))
    bias_spec = pl.BlockSpec((1, d), lambda i: (0, 0))
    adj_spec = pl.BlockSpec((bm, n), lambda i: (jnp.where(i < nb, i, i - nb), 0))
    out_spec = pl.BlockSpec((bm, d), lambda i: (jnp.where(i < nb, 0, i - nb), 0))

    return pl.pallas_call(
        body,
        grid=(2 * nb,),
        in_specs=[adj_spec, full_spec, sq_spec, bias_spec, sq_spec, sq_spec],
        out_specs=out_spec,
        out_shape=jax.ShapeDtypeStruct((n, d), jnp.float32),
        scratch_shapes=[pltpu.VMEM((n, d), jnp.float32)],
    )(adj, x, kernel_gate, bg, Weight_1, Weight_2)
